# baseline (speedup 1.0000x reference)
import functools

import jax
import jax.numpy as jnp
from jax import lax
from jax.experimental import pallas as pl
from jax.experimental.pallas import tpu as pltpu
from jax.experimental.pallas import tpu_sc as plsc

NC = 2
NS = 16
NW = NC * NS

CHUNK = 400
PACK_BC = 16384


def _tc_pack_table(table_t):
    d, v = table_t.shape
    bc = PACK_BC
    grid = (v + bc - 1) // bc
    tail = v - (grid - 1) * bc
    dn = (((0,), (0,)), ((), ()))
    eye = jnp.eye(d, dtype=jnp.float32)

    def pack_kernel(t_ref, e_ref, o_hbm, y0, y1, s0, s1):
        i = pl.program_id(0)

        def compute():
            yt = lax.dot_general(
                t_ref[...], e_ref[...], dn, preferred_element_type=jnp.float32)
            return jnp.concatenate([yt, jnp.zeros_like(yt)], axis=1)

        def dst(base, rows=bc):
            return o_hbm.at[pl.ds(base, rows), :]

        def ring(y, s, parity):
            @pl.when(jnp.logical_and(i != grid - 1, i % 2 == parity))
            def _():
                @pl.when(i >= 2)
                def _w():
                    pltpu.make_async_copy(y, dst(0), s).wait()
                y[...] = compute()
                pltpu.make_async_copy(y, dst(i * bc), s).start()

        ring(y0, s0, 0)
        ring(y1, s1, 1)

        @pl.when(i == grid - 1)
        def _tail():
            pltpu.make_async_copy(y0, dst(0), s0).wait()
            y0[...] = compute()
            cp = pltpu.make_async_copy(
                y0.at[pl.ds(0, tail)], dst((grid - 1) * bc, tail), s0)
            cp.start()
            cp.wait()
            pltpu.make_async_copy(y1, dst(0), s1).wait()

    return pl.pallas_call(
        pack_kernel,
        grid=(grid,),
        in_specs=[
            pl.BlockSpec((d, bc), lambda i: (0, i)),
            pl.BlockSpec((d, d), lambda i: (0, 0)),
        ],
        out_specs=pl.BlockSpec(memory_space=pl.ANY),
        out_shape=jax.ShapeDtypeStruct((v, 2 * d), jnp.float32),
        scratch_shapes=[
            pltpu.VMEM((bc, 2 * d), jnp.float32),
            pltpu.VMEM((bc, 2 * d), jnp.float32),
            pltpu.SemaphoreType.DMA,
            pltpu.SemaphoreType.DMA,
        ],
        compiler_params=pltpu.CompilerParams(
            fuse_transposed_lhs_in_matmul=True),
    )(table_t, eye)


def _sc_gather(table_z, ids, n):
    dz = table_z.shape[1]
    per_w = n // NW
    steps = per_w // CHUNK
    mesh = plsc.VectorSubcoreMesh(core_axis_name="c", subcore_axis_name="s")

    @functools.partial(
        pl.kernel,
        out_type=jax.ShapeDtypeStruct((n, dz), jnp.float32),
        mesh=mesh,
        scratch_types=[
            pltpu.VMEM((CHUNK,), jnp.int32),
            pltpu.VMEM((CHUNK,), jnp.int32),
            pltpu.VMEM((CHUNK, dz), jnp.float32),
            pltpu.VMEM((CHUNK, dz), jnp.float32),
            pltpu.SemaphoreType.DMA,
            pltpu.SemaphoreType.DMA,
            pltpu.SemaphoreType.DMA,
            pltpu.SemaphoreType.DMA,
        ],
        compiler_params=pltpu.CompilerParams(use_tc_tiling_on_sc=False),
    )
    def gather_kernel(tab, idx_hbm, out_hbm,
                      idx0, idx1, rows0, rows1, sg0, sg1, sw0, sw1):
        wid = lax.axis_index("s") * NC + lax.axis_index("c")
        base = wid * per_w
        idx = (idx0, idx1)
        rows = (rows0, rows1)
        sg = (sg0, sg1)
        sw = (sw0, sw1)

        def fire(c):
            p = c % 2
            pltpu.sync_copy(idx_hbm.at[pl.ds(base + c * CHUNK, CHUNK)], idx[p])
            return pltpu.async_copy(tab.at[idx[p]], rows[p], sg[p])

        g = [None] * steps
        w = [None] * steps
        g[0] = fire(0)
        for i in range(steps):
            p = i % 2
            if i + 1 < steps:
                if i >= 1:
                    w[i - 1].wait()
                g[i + 1] = fire(i + 1)
            g[i].wait()
            w[i] = pltpu.make_async_copy(
                rows[p], out_hbm.at[pl.ds(base + i * CHUNK, CHUNK)], sw[p])
            w[i].start()
        w[steps - 2].wait()
        w[steps - 1].wait()

    return gather_kernel(table_z, ids)


def _tc_project(x2, wz, b, bsz, seq, batch_block, block_off=0, out_prev=None):
    n, k = x2.shape
    h = wz.shape[1]
    rows_per_block = batch_block * seq

    def matmul_kernel(*refs):
        x_ref, w_ref, b_ref = refs[0], refs[1], refs[2]
        o_ref = refs[-1]
        y = jnp.dot(x_ref[...], w_ref[...],
                    preferred_element_type=jnp.float32) + b_ref[...]
        o_ref[...] = y.reshape(batch_block, seq, h)

    in_specs = [
        pl.BlockSpec((rows_per_block, k), lambda i: (i, 0)),
        pl.BlockSpec((k, h), lambda i: (0, 0)),
        pl.BlockSpec((1, h), lambda i: (0, 0)),
    ]
    args = [x2, wz, b.reshape(1, h)]
    kwargs = {}
    if out_prev is not None:
        in_specs.append(pl.BlockSpec(memory_space=pl.ANY))
        args.append(out_prev)
        kwargs["input_output_aliases"] = {3: 0}

    return pl.pallas_call(
        matmul_kernel,
        grid=(n // rows_per_block,),
        in_specs=in_specs,
        out_specs=pl.BlockSpec(
            (batch_block, seq, h), lambda i: (i + block_off, 0, 0)),
        out_shape=jax.ShapeDtypeStruct((bsz, seq, h), jnp.float32),
        **kwargs,
    )(*args)


def kernel(input_ids, token_embed, W, b):
    bsz, seq = input_ids.shape
    n = bsz * seq
    v, d = token_embed.shape
    h = W.shape[1]
    ids = input_ids.reshape(n).astype(jnp.int32)
    table_z = _tc_pack_table(token_embed.T)
    wz = jnp.concatenate([W, jnp.zeros_like(W)], axis=0)
    n2 = n // 2
    bb = 32
    xa = _sc_gather(table_z, ids[:n2], n2)
    xb = _sc_gather(table_z, ids[n2:], n2)
    out_a = _tc_project(xa, wz, b, bsz, seq, batch_block=bb)
    return _tc_project(xb, wz, b, bsz, seq, batch_block=bb,
                       block_off=bsz // (2 * bb), out_prev=out_a)


# speedup vs baseline: 1.0007x; 1.0007x over previous
import functools

import jax
import jax.numpy as jnp
from jax import lax
from jax.experimental import pallas as pl
from jax.experimental.pallas import tpu as pltpu
from jax.experimental.pallas import tpu_sc as plsc

NC = 2
NS = 16
NW = NC * NS

CHUNK = 400
PACK_BC = 16384


def _tc_pack_table(table_t):
    d, v = table_t.shape
    bc = PACK_BC
    grid = (v + bc - 1) // bc
    tail = v - (grid - 1) * bc
    dn = (((0,), (0,)), ((), ()))
    eye = jnp.eye(d, dtype=jnp.float32)

    def pack_kernel(t_ref, e_ref, o_hbm, y0, y1, s0, s1):
        i = pl.program_id(0)

        def compute():
            yt = lax.dot_general(
                t_ref[...], e_ref[...], dn, preferred_element_type=jnp.float32)
            return jnp.concatenate([yt, jnp.zeros_like(yt)], axis=1)

        def dst(base, rows=bc):
            return o_hbm.at[pl.ds(base, rows), :]

        def ring(y, s, parity):
            @pl.when(jnp.logical_and(i != grid - 1, i % 2 == parity))
            def _():
                @pl.when(i >= 2)
                def _w():
                    pltpu.make_async_copy(y, dst(0), s).wait()
                y[...] = compute()
                pltpu.make_async_copy(y, dst(i * bc), s).start()

        ring(y0, s0, 0)
        ring(y1, s1, 1)

        @pl.when(i == grid - 1)
        def _tail():
            pltpu.make_async_copy(y0, dst(0), s0).wait()
            y0[...] = compute()
            cp = pltpu.make_async_copy(
                y0.at[pl.ds(0, tail)], dst((grid - 1) * bc, tail), s0)
            cp.start()
            cp.wait()
            pltpu.make_async_copy(y1, dst(0), s1).wait()

    return pl.pallas_call(
        pack_kernel,
        grid=(grid,),
        in_specs=[
            pl.BlockSpec((d, bc), lambda i: (0, i)),
            pl.BlockSpec((d, d), lambda i: (0, 0)),
        ],
        out_specs=pl.BlockSpec(memory_space=pl.ANY),
        out_shape=jax.ShapeDtypeStruct((v, 2 * d), jnp.float32),
        scratch_shapes=[
            pltpu.VMEM((bc, 2 * d), jnp.float32),
            pltpu.VMEM((bc, 2 * d), jnp.float32),
            pltpu.SemaphoreType.DMA,
            pltpu.SemaphoreType.DMA,
        ],
        compiler_params=pltpu.CompilerParams(
            fuse_transposed_lhs_in_matmul=True),
    )(table_t, eye)


def _sc_gather(table_z, ids, n):
    dz = table_z.shape[1]
    per_w = n // NW
    steps = per_w // CHUNK
    mesh = plsc.VectorSubcoreMesh(core_axis_name="c", subcore_axis_name="s")

    @functools.partial(
        pl.kernel,
        out_type=jax.ShapeDtypeStruct((n, dz), jnp.float32),
        mesh=mesh,
        scratch_types=[
            pltpu.VMEM((CHUNK,), jnp.int32),
            pltpu.VMEM((CHUNK,), jnp.int32),
            pltpu.VMEM((CHUNK, dz), jnp.float32),
            pltpu.VMEM((CHUNK, dz), jnp.float32),
            pltpu.SemaphoreType.DMA,
            pltpu.SemaphoreType.DMA,
            pltpu.SemaphoreType.DMA,
            pltpu.SemaphoreType.DMA,
        ],
        compiler_params=pltpu.CompilerParams(use_tc_tiling_on_sc=False),
    )
    def gather_kernel(tab, idx_hbm, out_hbm,
                      idx0, idx1, rows0, rows1, sg0, sg1, sw0, sw1):
        wid = lax.axis_index("s") * NC + lax.axis_index("c")
        base = wid * per_w
        idx = (idx0, idx1)
        rows = (rows0, rows1)
        sg = (sg0, sg1)
        sw = (sw0, sw1)

        def fire(c):
            p = c % 2
            pltpu.sync_copy(idx_hbm.at[pl.ds(base + c * CHUNK, CHUNK)], idx[p])
            return pltpu.async_copy(tab.at[idx[p]], rows[p], sg[p])

        g = [None] * steps
        w = [None] * steps
        g[0] = fire(0)
        for i in range(steps):
            p = i % 2
            if i + 1 < steps:
                if i >= 1:
                    w[i - 1].wait()
                g[i + 1] = fire(i + 1)
            g[i].wait()
            w[i] = pltpu.make_async_copy(
                rows[p], out_hbm.at[pl.ds(base + i * CHUNK, CHUNK)], sw[p])
            w[i].start()
        w[steps - 2].wait()
        w[steps - 1].wait()

    return gather_kernel(table_z, ids)


def _tc_project(x2, wz, b, bsz, seq, batch_block, block_off=0, out_prev=None):
    n, k = x2.shape
    h = wz.shape[1]
    rows_per_block = batch_block * seq

    def matmul_kernel(*refs):
        x_ref, w_ref, b_ref = refs[0], refs[1], refs[2]
        o_ref = refs[-1]
        y = jnp.dot(x_ref[...], w_ref[...],
                    preferred_element_type=jnp.float32) + b_ref[...]
        o_ref[...] = y.reshape(batch_block, seq, h)

    in_specs = [
        pl.BlockSpec((rows_per_block, k), lambda i: (i, 0)),
        pl.BlockSpec((k, h), lambda i: (0, 0)),
        pl.BlockSpec((1, h), lambda i: (0, 0)),
    ]
    args = [x2, wz, b.reshape(1, h)]
    kwargs = {}
    if out_prev is not None:
        in_specs.append(pl.BlockSpec(memory_space=pl.ANY))
        args.append(out_prev)
        kwargs["input_output_aliases"] = {3: 0}

    return pl.pallas_call(
        matmul_kernel,
        grid=(n // rows_per_block,),
        in_specs=in_specs,
        out_specs=pl.BlockSpec(
            (batch_block, seq, h), lambda i: (i + block_off, 0, 0)),
        out_shape=jax.ShapeDtypeStruct((bsz, seq, h), jnp.float32),
        **kwargs,
    )(*args)


def kernel(input_ids, token_embed, W, b):
    bsz, seq = input_ids.shape
    n = bsz * seq
    v, d = token_embed.shape
    h = W.shape[1]
    ids = input_ids.reshape(n).astype(jnp.int32)
    table_z = _tc_pack_table(token_embed.T)
    wz = jnp.concatenate([W, jnp.zeros_like(W)], axis=0)
    x2 = _sc_gather(table_z, ids, n)
    return _tc_project(x2, wz, b, bsz, seq, batch_block=32)
